# Initial kernel scaffold; baseline (speedup 1.0000x reference)
import jax
import jax.numpy as jnp
from jax.experimental import pallas as pl


def kernel(input_ids, lut, base_idx, fine_idx):
    raise NotImplementedError("write your pallas kernel here")


# speedup vs baseline: 99.7886x; 99.7886x over previous
import functools

import jax
import jax.numpy as jnp
from jax import lax
from jax.experimental import pallas as pl
from jax.experimental.pallas import tpu as pltpu
from jax.experimental.pallas import tpu_sc as plsc

DIM = 64
LUT_SIZE = 65536
NUM_CORES = 2
NUM_SUBCORES = 16
NUM_WORKERS = NUM_CORES * NUM_SUBCORES
LANES = 16
CHUNK = 128


def _sc_body(ids_hbm, lut_hbm, base_hbm, fine_hbm, out_hbm,
             ids_v, lut_v, base_v, fine_v, out_v, sem_a, sem_b):
    bpw = ids_v.shape[0]
    n_chunks = bpw // CHUNK
    wid = lax.axis_index("s") * NUM_CORES + lax.axis_index("c")
    tok0 = wid * bpw

    pltpu.sync_copy(ids_hbm.at[pl.ds(tok0, bpw)], ids_v)
    pltpu.sync_copy(lut_hbm, lut_v)

    def chunk_body(ci, carry):
        idx_slice = ids_v.at[pl.ds(ci * CHUNK, CHUNK)]
        cp_a = pltpu.async_copy(base_hbm.at[idx_slice], base_v, sem_a)
        cp_b = pltpu.async_copy(fine_hbm.at[idx_slice], fine_v, sem_b)
        cp_a.wait()
        cp_b.wait()

        def tok_body(t, carry2):
            for j in range(DIM // LANES):
                sl = pl.ds(j * LANES, LANES)
                b = base_v[t, sl]
                f = fine_v[t, sl]
                c = b * 256 + f
                out_v[t, sl] = plsc.load_gather(lut_v, [c])
            return carry2

        lax.fori_loop(0, CHUNK, tok_body, 0, unroll=2)
        pltpu.sync_copy(out_v, out_hbm.at[pl.ds(tok0 + ci * CHUNK, CHUNK)])
        return carry

    lax.fori_loop(0, n_chunks, chunk_body, 0)


@functools.partial(jax.jit, static_argnames=())
def _sc_embed(ids_flat, lut, base_idx, fine_idx):
    n_tok = ids_flat.shape[0]
    bpw = n_tok // NUM_WORKERS
    mesh = plsc.VectorSubcoreMesh(core_axis_name="c", subcore_axis_name="s")
    kern = pl.kernel(
        _sc_body,
        out_type=jax.ShapeDtypeStruct((n_tok, DIM), jnp.float32),
        mesh=mesh,
        scratch_types=[
            pltpu.VMEM((bpw,), jnp.int32),
            pltpu.VMEM((LUT_SIZE,), jnp.float32),
            pltpu.VMEM((CHUNK, DIM), jnp.int32),
            pltpu.VMEM((CHUNK, DIM), jnp.int32),
            pltpu.VMEM((CHUNK, DIM), jnp.float32),
            pltpu.SemaphoreType.DMA,
            pltpu.SemaphoreType.DMA,
        ],
        compiler_params=pltpu.CompilerParams(
            needs_layout_passes=False, use_tc_tiling_on_sc=False),
    )
    return kern(ids_flat, lut, base_idx, fine_idx)


def kernel(input_ids, lut, base_idx, fine_idx):
    ids_flat = input_ids.reshape(-1).astype(jnp.int32)
    out = _sc_embed(ids_flat, lut, base_idx.astype(jnp.int32),
                    fine_idx.astype(jnp.int32))
    return out.reshape(input_ids.shape + (DIM,))


# speedup vs baseline: 188.5409x; 1.8894x over previous
import functools

import jax
import jax.numpy as jnp
from jax import lax
from jax.experimental import pallas as pl
from jax.experimental.pallas import tpu as pltpu
from jax.experimental.pallas import tpu_sc as plsc

DIM = 64
LUT_SIZE = 65536
NUM_CORES = 2
NUM_SUBCORES = 16
NUM_WORKERS = NUM_CORES * NUM_SUBCORES
LANES = 16
CHUNK = 128
NBUF = 2


def _sc_body(ids_hbm, lut_hbm, base_hbm, fine_hbm, out_hbm,
             ids_v, lut_v, base_v, fine_v, out_v,
             sems_b, sems_f, sems_o):
    bpw = ids_v.shape[0]
    n_chunks = bpw // CHUNK
    wid = lax.axis_index("s") * NUM_CORES + lax.axis_index("c")
    tok0 = wid * bpw

    pltpu.sync_copy(ids_hbm.at[pl.ds(tok0, bpw)], ids_v)
    pltpu.sync_copy(lut_hbm, lut_v)

    def issue(ci, slot):
        idx = ids_v.at[pl.ds(ci * CHUNK, CHUNK)]
        pltpu.async_copy(base_hbm.at[idx], base_v.at[slot], sems_b[slot])
        pltpu.async_copy(fine_hbm.at[idx], fine_v.at[slot], sems_f[slot])

    def wait_in(slot):
        idx = ids_v.at[pl.ds(0, CHUNK)]
        pltpu.make_async_copy(base_hbm.at[idx], base_v.at[slot], sems_b[slot]).wait()
        pltpu.make_async_copy(fine_hbm.at[idx], fine_v.at[slot], sems_f[slot]).wait()

    def wait_out(slot):
        pltpu.make_async_copy(
            out_v.at[slot], out_hbm.at[pl.ds(0, CHUNK)], sems_o[slot]).wait()

    issue(0, 0)

    def outer(g, carry):
        for b in range(NBUF):
            cur = g * NBUF + b
            nxt = cur + 1

            @pl.when(nxt < n_chunks)
            def _():
                issue(nxt, (b + 1) % NBUF)

            wait_in(b)

            @pl.when(cur >= NBUF)
            def _():
                wait_out(b)

            @plsc.parallel_loop(0, CHUNK, 1, unroll=4)
            def _(t):
                for j in range(DIM // LANES):
                    sl = pl.ds(j * LANES, LANES)
                    c = base_v[b, t, sl] * 256 + fine_v[b, t, sl]
                    out_v[b, t, sl] = plsc.load_gather(lut_v, [c])

            pltpu.async_copy(
                out_v.at[b],
                out_hbm.at[pl.ds(tok0 + cur * CHUNK, CHUNK)],
                sems_o[b])
        return carry

    lax.fori_loop(0, n_chunks // NBUF, outer, 0)
    for b in range(NBUF):
        wait_out(b)


@jax.jit
def _sc_embed(ids_flat, lut, base_idx, fine_idx):
    n_tok = ids_flat.shape[0]
    bpw = n_tok // NUM_WORKERS
    mesh = plsc.VectorSubcoreMesh(core_axis_name="c", subcore_axis_name="s")
    kern = pl.kernel(
        _sc_body,
        out_type=jax.ShapeDtypeStruct((n_tok, DIM), jnp.float32),
        mesh=mesh,
        scratch_types=[
            pltpu.VMEM((bpw,), jnp.int32),
            pltpu.VMEM((LUT_SIZE,), jnp.float32),
            pltpu.VMEM((NBUF, CHUNK, DIM), jnp.int32),
            pltpu.VMEM((NBUF, CHUNK, DIM), jnp.int32),
            pltpu.VMEM((NBUF, CHUNK, DIM), jnp.float32),
            [pltpu.SemaphoreType.DMA] * NBUF,
            [pltpu.SemaphoreType.DMA] * NBUF,
            [pltpu.SemaphoreType.DMA] * NBUF,
        ],
        compiler_params=pltpu.CompilerParams(
            needs_layout_passes=False, use_tc_tiling_on_sc=False),
    )
    return kern(ids_flat, lut, base_idx, fine_idx)


def kernel(input_ids, lut, base_idx, fine_idx):
    ids_flat = input_ids.reshape(-1).astype(jnp.int32)
    out = _sc_embed(ids_flat, lut, base_idx.astype(jnp.int32),
                    fine_idx.astype(jnp.int32))
    return out.reshape(input_ids.shape + (DIM,))
